# baseline (speedup 1.0000x reference)
import jax
import jax.numpy as jnp
from jax.experimental import pallas as pl

_T = 100000
_C = 8
_WINDOW = 100
_STRIDE = 50
_SIG = 584
_EMBED = 64
_NTOK = 1024
_NWIN = (_T - _WINDOW) // _STRIDE + 1
_BLK = 256
_GRID = 8
_NCHUNK = _GRID * _BLK + 8


def _sig_vq_kernel(ca_ref, cb2_ref, w_ref, b_ref, code_ref, tok_ref, z_ref):
    ina = ca_ref[...].transpose(0, 2, 1)
    inb = cb2_ref[...].transpose(0, 2, 1)
    nw = ina.shape[0]
    xb = jnp.concatenate([ina[1:], inb[0:1]], axis=0)
    zpad = jnp.zeros((nw, _C, 29), jnp.float32)
    xcur = jnp.concatenate([ina, xb[:, :, :49], zpad], axis=2)
    xnxt = jnp.concatenate([ina[:, :, 1:], xb, zpad], axis=2)

    v = xnxt - xcur
    p = xcur - ina[:, :, 0:1]
    s = xb[:, :, 49:50] - xnxt

    u = p + 0.5 * v
    q = (v[:, :, None, :] * v[:, None, :, :]).reshape(nw, 64, 128)
    m = (u[:, :, None, :] * v[:, None, :, :]).reshape(nw, 64, 128)

    def bdot(a, b):
        return jax.lax.dot_general(
            a, b, (((2,), (2,)), ((0,), (0,))),
            precision=jax.lax.Precision.HIGHEST,
            preferred_element_type=jnp.float32)

    out1 = bdot(v / 6.0 + 0.5 * p, q)
    rhs2 = jnp.concatenate([s, jnp.ones((nw, 8, 128), jnp.float32)], axis=1)
    out2 = bdot(m, rhs2)

    s1 = xb[:, :, 49] - ina[:, :, 0]
    s2 = out2[:, :, 8]
    s3 = out1.reshape(nw, 512) + out2[:, :, 0:8].reshape(nw, 512)
    sigs = jnp.concatenate([s1, s2, s3], axis=1)

    z = jax.lax.dot_general(
        sigs, w_ref[...], (((1,), (1,)), ((), ())),
        preferred_element_type=jnp.float32) + b_ref[...]

    code = code_ref[...]
    cc = jnp.sum(code * code, axis=1)
    zz = jnp.sum(z * z, axis=1, keepdims=True)
    d2 = zz + cc[None, :] - 2.0 * jax.lax.dot_general(
        z, code, (((1,), (1,)), ((), ())),
        preferred_element_type=jnp.float32)

    dmin = jnp.min(d2, axis=1, keepdims=True)
    ids = jax.lax.broadcasted_iota(jnp.int32, d2.shape, 1)
    tok = jnp.min(jnp.where(d2 <= dmin, ids, jnp.int32(_NTOK)), axis=1)

    tok_ref[0, 0, :] = tok
    z_ref[...] = z


def kernel(data, W, b, codebook):
    pad_rows = _NCHUNK * _STRIDE - _T
    chunks = jnp.concatenate(
        [data, jnp.zeros((pad_rows, _C), data.dtype)], axis=0)
    chunks = chunks.reshape(_NCHUNK, _STRIDE, _C)

    tok, z = pl.pallas_call(
        _sig_vq_kernel,
        grid=(_GRID,),
        in_specs=[
            pl.BlockSpec((_BLK, _STRIDE, _C), lambda i: (i, 0, 0)),
            pl.BlockSpec((8, _STRIDE, _C), lambda i: ((i + 1) * (_BLK // 8), 0, 0)),
            pl.BlockSpec((_EMBED, _SIG), lambda i: (0, 0)),
            pl.BlockSpec((1, _EMBED), lambda i: (0, 0)),
            pl.BlockSpec((_NTOK, _EMBED), lambda i: (0, 0)),
        ],
        out_specs=[
            pl.BlockSpec((1, 1, _BLK), lambda i: (i, 0, 0)),
            pl.BlockSpec((_BLK, _EMBED), lambda i: (i, 0)),
        ],
        out_shape=[
            jax.ShapeDtypeStruct((_GRID, 1, _BLK), jnp.int32),
            jax.ShapeDtypeStruct((_GRID * _BLK, _EMBED), jnp.float32),
        ],
    )(chunks, chunks, W, b.reshape(1, _EMBED), codebook)

    return tok.reshape(_GRID * _BLK)[:_NWIN], z[:_NWIN]


# speedup vs baseline: 1.4856x; 1.4856x over previous
import jax
import jax.numpy as jnp
from jax.experimental import pallas as pl

_T = 100000
_C = 8
_WINDOW = 100
_STRIDE = 50
_SIG = 584
_EMBED = 64
_NTOK = 1024
_NWIN = (_T - _WINDOW) // _STRIDE + 1
_BLK = 256
_GRID = 8
_NCHUNK = _GRID * _BLK + 8


def _sig_vq_kernel(ca_ref, cb2_ref, w_ref, b_ref, code_ref, tok_ref, z_ref):
    ina = ca_ref[...]
    inb = cb2_ref[...]
    nw = ina.shape[0]
    xb = jnp.concatenate([ina[1:], inb[0:1]], axis=0)
    zpad = jnp.zeros((nw, _C, 29), jnp.float32)
    xcur = jnp.concatenate([ina, xb[:, :, :49], zpad], axis=2)
    xnxt = jnp.concatenate([ina[:, :, 1:], xb, zpad], axis=2)

    v = xnxt - xcur
    p = xcur - ina[:, :, 0:1]
    s = xb[:, :, 49:50] - xnxt

    u = p + 0.5 * v
    c = v / 6.0 + 0.5 * p
    lhs = jnp.concatenate([
        (c[:, :, None, :] * v[:, None, :, :]).reshape(nw, 64, 128),
        (u[:, :, None, :] * v[:, None, :, :]).reshape(nw, 64, 128)], axis=2)
    rhs = jnp.concatenate([
        jnp.concatenate([v, s], axis=2),
        jnp.concatenate([jnp.zeros((nw, 8, 128), jnp.float32),
                         jnp.ones((nw, 8, 128), jnp.float32)], axis=2)], axis=1)
    out = jax.lax.dot_general(
        lhs, rhs, (((2,), (2,)), ((0,), (0,))),
        precision=jax.lax.Precision.HIGHEST,
        preferred_element_type=jnp.float32)

    s1 = xb[:, :, 49] - ina[:, :, 0]
    s2 = out[:, :, 8]
    s3 = out[:, :, 0:8].reshape(nw, 512)
    sigs = jnp.concatenate([s1, s2, s3], axis=1)

    z = jax.lax.dot_general(
        sigs, w_ref[...], (((1,), (1,)), ((), ())),
        preferred_element_type=jnp.float32) + b_ref[...]

    code = code_ref[...]
    cc = jnp.sum(code * code, axis=1)
    zz = jnp.sum(z * z, axis=1, keepdims=True)
    d2 = zz + cc[None, :] - 2.0 * jax.lax.dot_general(
        z, code, (((1,), (1,)), ((), ())),
        preferred_element_type=jnp.float32)

    dmin = jnp.min(d2, axis=1, keepdims=True)
    ids = jax.lax.broadcasted_iota(jnp.int32, d2.shape, 1)
    tok = jnp.min(jnp.where(d2 <= dmin, ids, jnp.int32(_NTOK)), axis=1)

    tok_ref[0, 0, :] = tok
    z_ref[...] = z


def kernel(data, W, b, codebook):
    pad_rows = _NCHUNK * _STRIDE - _T
    chunks = jnp.concatenate(
        [data, jnp.zeros((pad_rows, _C), data.dtype)], axis=0)
    chunks = chunks.reshape(_NCHUNK, _STRIDE, _C).transpose(0, 2, 1)

    tok, z = pl.pallas_call(
        _sig_vq_kernel,
        grid=(_GRID,),
        in_specs=[
            pl.BlockSpec((_BLK, _C, _STRIDE), lambda i: (i, 0, 0)),
            pl.BlockSpec((8, _C, _STRIDE), lambda i: ((i + 1) * (_BLK // 8), 0, 0)),
            pl.BlockSpec((_EMBED, _SIG), lambda i: (0, 0)),
            pl.BlockSpec((1, _EMBED), lambda i: (0, 0)),
            pl.BlockSpec((_NTOK, _EMBED), lambda i: (0, 0)),
        ],
        out_specs=[
            pl.BlockSpec((1, 1, _BLK), lambda i: (i, 0, 0)),
            pl.BlockSpec((_BLK, _EMBED), lambda i: (i, 0)),
        ],
        out_shape=[
            jax.ShapeDtypeStruct((_GRID, 1, _BLK), jnp.int32),
            jax.ShapeDtypeStruct((_GRID * _BLK, _EMBED), jnp.float32),
        ],
    )(chunks, chunks, W, b.reshape(1, _EMBED), codebook)

    return tok.reshape(_GRID * _BLK)[:_NWIN], z[:_NWIN]
